# baseline (speedup 1.0000x reference)
import jax
import jax.numpy as jnp
from jax.experimental import pallas as pl
from jax.experimental.pallas import tpu as pltpu

N = 262144
D = 64
CB = 8192
TARGET_IDX = 0
TEST = 0.0

VR = 2048
VC = N // VR


def _threefry_bits(rows_u32):
    ks0 = jnp.uint32(0)
    ks1 = jnp.uint32(42)
    ks2 = jnp.uint32(0x1BD11BDA) ^ ks0 ^ ks1
    ks = (ks0, ks1, ks2)
    rotations = ((13, 15, 26, 6), (17, 29, 16, 24))
    x0 = jnp.zeros_like(rows_u32) + ks0
    x1 = rows_u32 + ks1
    for i in range(5):
        for r in rotations[i % 2]:
            x0 = x0 + x1
            x1 = (x1 << jnp.uint32(r)) | (x1 >> jnp.uint32(32 - r))
            x1 = x0 ^ x1
        x0 = x0 + ks[(i + 1) % 3]
        x1 = x1 + ks[(i + 2) % 3] + jnp.uint32(i + 1)
    return x0 ^ x1


def _p_left(tc, td):
    lb = tc - td
    rb = tc + td
    test = jnp.float32(TEST)
    cross = jnp.logical_and(rb > test, lb <= test)
    denom = jnp.where(cross, rb - lb, jnp.float32(1.0))
    p_left = jnp.where(rb <= test, jnp.float32(1.0),
                       jnp.where(lb > test, jnp.float32(0.0),
                                 (test - lb) / denom))
    return jnp.clip(p_left, 0.0, 1.0)


def _new_cols(tc, td):
    test = jnp.float32(TEST)
    lb = tc - td
    rb = tc + td
    min_rt = jnp.minimum(rb, test)
    new_lc = (lb + min_rt) * jnp.float32(0.5)
    new_ld = (min_rt - lb) * jnp.float32(0.5)
    max_lt = jnp.maximum(lb, test)
    new_rc = (max_lt + rb) * jnp.float32(0.5)
    new_rd = (rb - max_lt) * jnp.float32(0.5)
    return new_lc, new_ld, new_rc, new_rd


def _vec_kernel(tc_ref, td_ref, p_ref,
                lp_ref, rp_ref, left_ref, right_ref):
    p_left = _p_left(tc_ref[...], td_ref[...])

    rows = (jax.lax.broadcasted_iota(jnp.uint32, (VR, VC), 0) * jnp.uint32(VC)
            + jax.lax.broadcasted_iota(jnp.uint32, (VR, VC), 1))
    bits = _threefry_bits(rows)
    fbits = (bits >> jnp.uint32(9)) | jnp.uint32(0x3F800000)
    u = jax.lax.bitcast_convert_type(fbits, jnp.float32) - jnp.float32(1.0)
    left = (u < p_left).astype(jnp.float32)

    pv = p_ref[...]
    lp_ref[...] = pv + jnp.log(jnp.maximum(p_left, jnp.float32(1e-12)))
    rp_ref[...] = pv + jnp.log(jnp.maximum(jnp.float32(1.0) - p_left,
                                           jnp.float32(1e-12)))
    left_ref[...] = left
    right_ref[...] = jnp.float32(1.0) - left


def _copy_kernel(ct_ref, dt_ref, lc_ref, ld_ref, rc_ref, rd_ref):
    cb = ct_ref[...]
    db = dt_ref[...]
    tc = cb[0:1, :]
    td = db[0:1, :]
    new_lc, new_ld, new_rc, new_rd = _new_cols(tc, td)
    row0 = jax.lax.broadcasted_iota(jnp.int32, (D, CB), 0) == TARGET_IDX
    lc_ref[...] = jnp.where(row0, new_lc, cb)
    ld_ref[...] = jnp.where(row0, new_ld, db)
    rc_ref[...] = jnp.where(row0, new_rc, cb)
    rd_ref[...] = jnp.where(row0, new_rd, db)


def kernel(c, delta, p):
    tc = c[:, TARGET_IDX].reshape(VR, VC)
    td = delta[:, TARGET_IDX].reshape(VR, VC)
    p2 = p.reshape(VR, VC)

    vec_out = jax.ShapeDtypeStruct((VR, VC), jnp.float32)
    whole = pl.BlockSpec((VR, VC), lambda: (0, 0))
    logp_left, logp_right, left, right = pl.pallas_call(
        _vec_kernel,
        grid=(),
        in_specs=[whole, whole, whole],
        out_specs=[whole, whole, whole, whole],
        out_shape=[vec_out, vec_out, vec_out, vec_out],
    )(tc, td, p2)
    logp_left = logp_left.reshape(N)
    logp_right = logp_right.reshape(N)
    left = left.reshape(N)
    right = right.reshape(N)

    ct = c.T
    dt = delta.T
    mat_spec = pl.BlockSpec((D, CB), lambda i: (0, i))
    mat_out = jax.ShapeDtypeStruct((D, N), jnp.float32)
    xlc, xld, xrc, xrd = pl.pallas_call(
        _copy_kernel,
        grid=(N // CB,),
        in_specs=[mat_spec, mat_spec],
        out_specs=[mat_spec, mat_spec, mat_spec, mat_spec],
        out_shape=[mat_out, mat_out, mat_out, mat_out],
        compiler_params=pltpu.CompilerParams(
            dimension_semantics=("parallel",)),
    )(ct, dt)

    return (xlc.T, xld.T, logp_left, xrc.T, xrd.T, logp_right, left, right)


# speedup vs baseline: 1.0313x; 1.0313x over previous
import jax
import jax.numpy as jnp
from jax.experimental import pallas as pl
from jax.experimental.pallas import tpu as pltpu

N = 262144
D = 64
CB = 16384
TARGET_IDX = 0
TEST = 0.0

VR = 2048
VC = N // VR


def _threefry_bits(rows_u32):
    ks0 = jnp.uint32(0)
    ks1 = jnp.uint32(42)
    ks2 = jnp.uint32(0x1BD11BDA) ^ ks0 ^ ks1
    ks = (ks0, ks1, ks2)
    rotations = ((13, 15, 26, 6), (17, 29, 16, 24))
    x0 = jnp.zeros_like(rows_u32) + ks0
    x1 = rows_u32 + ks1
    for i in range(5):
        for r in rotations[i % 2]:
            x0 = x0 + x1
            x1 = (x1 << jnp.uint32(r)) | (x1 >> jnp.uint32(32 - r))
            x1 = x0 ^ x1
        x0 = x0 + ks[(i + 1) % 3]
        x1 = x1 + ks[(i + 2) % 3] + jnp.uint32(i + 1)
    return x0 ^ x1


def _p_left(tc, td):
    lb = tc - td
    rb = tc + td
    test = jnp.float32(TEST)
    cross = jnp.logical_and(rb > test, lb <= test)
    denom = jnp.where(cross, rb - lb, jnp.float32(1.0))
    p_left = jnp.where(rb <= test, jnp.float32(1.0),
                       jnp.where(lb > test, jnp.float32(0.0),
                                 (test - lb) / denom))
    return jnp.clip(p_left, 0.0, 1.0)


def _new_cols(tc, td):
    test = jnp.float32(TEST)
    lb = tc - td
    rb = tc + td
    min_rt = jnp.minimum(rb, test)
    new_lc = (lb + min_rt) * jnp.float32(0.5)
    new_ld = (min_rt - lb) * jnp.float32(0.5)
    max_lt = jnp.maximum(lb, test)
    new_rc = (max_lt + rb) * jnp.float32(0.5)
    new_rd = (rb - max_lt) * jnp.float32(0.5)
    return new_lc, new_ld, new_rc, new_rd


def _vec_kernel(tc_ref, td_ref, p_ref,
                lp_ref, rp_ref, left_ref, right_ref):
    p_left = _p_left(tc_ref[...], td_ref[...])

    rows = (jax.lax.broadcasted_iota(jnp.uint32, (VR, VC), 0) * jnp.uint32(VC)
            + jax.lax.broadcasted_iota(jnp.uint32, (VR, VC), 1))
    bits = _threefry_bits(rows)
    fbits = (bits >> jnp.uint32(9)) | jnp.uint32(0x3F800000)
    u = jax.lax.bitcast_convert_type(fbits, jnp.float32) - jnp.float32(1.0)
    left = (u < p_left).astype(jnp.float32)

    pv = p_ref[...]
    lp_ref[...] = pv + jnp.log(jnp.maximum(p_left, jnp.float32(1e-12)))
    rp_ref[...] = pv + jnp.log(jnp.maximum(jnp.float32(1.0) - p_left,
                                           jnp.float32(1e-12)))
    left_ref[...] = left
    right_ref[...] = jnp.float32(1.0) - left


def _copy_kernel(ct_ref, dt_ref, lc_ref, ld_ref, rc_ref, rd_ref):
    cb = ct_ref[...]
    db = dt_ref[...]
    tc = cb[0:1, :]
    td = db[0:1, :]
    new_lc, new_ld, new_rc, new_rd = _new_cols(tc, td)
    row0 = jax.lax.broadcasted_iota(jnp.int32, (D, CB), 0) == TARGET_IDX
    lc_ref[...] = jnp.where(row0, new_lc, cb)
    ld_ref[...] = jnp.where(row0, new_ld, db)
    rc_ref[...] = jnp.where(row0, new_rc, cb)
    rd_ref[...] = jnp.where(row0, new_rd, db)


def kernel(c, delta, p):
    tc = c[:, TARGET_IDX].reshape(VR, VC)
    td = delta[:, TARGET_IDX].reshape(VR, VC)
    p2 = p.reshape(VR, VC)

    vec_out = jax.ShapeDtypeStruct((VR, VC), jnp.float32)
    whole = pl.BlockSpec((VR, VC), lambda: (0, 0))
    logp_left, logp_right, left, right = pl.pallas_call(
        _vec_kernel,
        grid=(),
        in_specs=[whole, whole, whole],
        out_specs=[whole, whole, whole, whole],
        out_shape=[vec_out, vec_out, vec_out, vec_out],
    )(tc, td, p2)
    logp_left = logp_left.reshape(N)
    logp_right = logp_right.reshape(N)
    left = left.reshape(N)
    right = right.reshape(N)

    ct = c.T
    dt = delta.T
    mat_spec = pl.BlockSpec((D, CB), lambda i: (0, i))
    mat_out = jax.ShapeDtypeStruct((D, N), jnp.float32)
    xlc, xld, xrc, xrd = pl.pallas_call(
        _copy_kernel,
        grid=(N // CB,),
        in_specs=[mat_spec, mat_spec],
        out_specs=[mat_spec, mat_spec, mat_spec, mat_spec],
        out_shape=[mat_out, mat_out, mat_out, mat_out],
        compiler_params=pltpu.CompilerParams(
            dimension_semantics=("parallel",)),
    )(ct, dt)

    return (xlc.T, xld.T, logp_left, xrc.T, xrd.T, logp_right, left, right)


# speedup vs baseline: 1.1601x; 1.1248x over previous
import jax
import jax.numpy as jnp
from jax.experimental import pallas as pl
from jax.experimental.pallas import tpu as pltpu

N = 262144
D = 64
CB = 16384
TARGET_IDX = 0
TEST = 0.0

VC = 128
VR = N // VC
CBR = CB // VC


def _threefry_bits(rows_u32):
    ks0 = jnp.uint32(0)
    ks1 = jnp.uint32(42)
    ks2 = jnp.uint32(0x1BD11BDA) ^ ks0 ^ ks1
    ks = (ks0, ks1, ks2)
    rotations = ((13, 15, 26, 6), (17, 29, 16, 24))
    x0 = jnp.zeros_like(rows_u32) + ks0
    x1 = rows_u32 + ks1
    for i in range(5):
        for r in rotations[i % 2]:
            x0 = x0 + x1
            x1 = (x1 << jnp.uint32(r)) | (x1 >> jnp.uint32(32 - r))
            x1 = x0 ^ x1
        x0 = x0 + ks[(i + 1) % 3]
        x1 = x1 + ks[(i + 2) % 3] + jnp.uint32(i + 1)
    return x0 ^ x1


def _p_left(tc, td):
    lb = tc - td
    rb = tc + td
    test = jnp.float32(TEST)
    cross = jnp.logical_and(rb > test, lb <= test)
    denom = jnp.where(cross, rb - lb, jnp.float32(1.0))
    p_left = jnp.where(rb <= test, jnp.float32(1.0),
                       jnp.where(lb > test, jnp.float32(0.0),
                                 (test - lb) / denom))
    return jnp.clip(p_left, 0.0, 1.0)


def _new_cols(tc, td):
    test = jnp.float32(TEST)
    lb = tc - td
    rb = tc + td
    min_rt = jnp.minimum(rb, test)
    new_lc = (lb + min_rt) * jnp.float32(0.5)
    new_ld = (min_rt - lb) * jnp.float32(0.5)
    max_lt = jnp.maximum(lb, test)
    new_rc = (max_lt + rb) * jnp.float32(0.5)
    new_rd = (rb - max_lt) * jnp.float32(0.5)
    return new_lc, new_ld, new_rc, new_rd


def _fused_kernel(ct_ref, dt_ref, p_ref,
                  lc_ref, ld_ref, rc_ref, rd_ref,
                  lp_ref, rp_ref, left_ref, right_ref):
    cb = ct_ref[...]
    db = dt_ref[...]
    tc = cb[0:1, :]
    td = db[0:1, :]
    new_lc, new_ld, new_rc, new_rd = _new_cols(tc, td)
    row0 = jax.lax.broadcasted_iota(jnp.int32, (D, CB), 0) == TARGET_IDX
    lc_ref[...] = jnp.where(row0, new_lc, cb)
    ld_ref[...] = jnp.where(row0, new_ld, db)
    rc_ref[...] = jnp.where(row0, new_rc, cb)
    rd_ref[...] = jnp.where(row0, new_rd, db)

    tc2 = tc.reshape(CBR, VC)
    td2 = td.reshape(CBR, VC)
    p_left = _p_left(tc2, td2)
    base = (pl.program_id(0) * CB).astype(jnp.uint32)
    rows = (base
            + jax.lax.broadcasted_iota(jnp.uint32, (CBR, VC), 0)
            * jnp.uint32(VC)
            + jax.lax.broadcasted_iota(jnp.uint32, (CBR, VC), 1))
    bits = _threefry_bits(rows)
    fbits = (bits >> jnp.uint32(9)) | jnp.uint32(0x3F800000)
    u = jax.lax.bitcast_convert_type(fbits, jnp.float32) - jnp.float32(1.0)
    left = (u < p_left).astype(jnp.float32)

    pv = p_ref[...]
    lp_ref[...] = pv + jnp.log(jnp.maximum(p_left, jnp.float32(1e-12)))
    rp_ref[...] = pv + jnp.log(jnp.maximum(jnp.float32(1.0) - p_left,
                                           jnp.float32(1e-12)))
    left_ref[...] = left
    right_ref[...] = jnp.float32(1.0) - left


def kernel(c, delta, p):
    ct = c.T
    dt = delta.T
    p2 = p.reshape(VR, VC)

    mat_spec = pl.BlockSpec((D, CB), lambda i: (0, i))
    vec_spec = pl.BlockSpec((CBR, VC), lambda i: (i, 0))
    mat_out = jax.ShapeDtypeStruct((D, N), jnp.float32)
    vec_out = jax.ShapeDtypeStruct((VR, VC), jnp.float32)
    xlc, xld, xrc, xrd, lp, rp, left, right = pl.pallas_call(
        _fused_kernel,
        grid=(N // CB,),
        in_specs=[mat_spec, mat_spec, vec_spec],
        out_specs=[mat_spec, mat_spec, mat_spec, mat_spec,
                   vec_spec, vec_spec, vec_spec, vec_spec],
        out_shape=[mat_out, mat_out, mat_out, mat_out,
                   vec_out, vec_out, vec_out, vec_out],
        compiler_params=pltpu.CompilerParams(
            dimension_semantics=("parallel",)),
    )(ct, dt, p2)

    return (xlc.T, xld.T, lp.reshape(N), xrc.T, xrd.T, rp.reshape(N),
            left.reshape(N), right.reshape(N))
